# Initial kernel scaffold; baseline (speedup 1.0000x reference)
import jax
import jax.numpy as jnp
from jax.experimental import pallas as pl


def kernel(x, pos_emb):
    raise NotImplementedError("write your pallas kernel here")


# speedup vs baseline: 1.9592x; 1.9592x over previous
import jax
import jax.numpy as jnp
from jax.experimental import pallas as pl


def _add_kernel(x_ref, pe_ref, o_ref):
    o_ref[...] = x_ref[...] + pe_ref[...]


def kernel(x, pos_emb):
    b, s, d = x.shape
    pe = pos_emb[:s]
    s_blk = 1024
    grid = (s // s_blk, b)
    return pl.pallas_call(
        _add_kernel,
        grid=grid,
        in_specs=[
            pl.BlockSpec((1, s_blk, d), lambda i, j: (j, i, 0)),
            pl.BlockSpec((s_blk, d), lambda i, j: (i, 0)),
        ],
        out_specs=pl.BlockSpec((1, s_blk, d), lambda i, j: (j, i, 0)),
        out_shape=jax.ShapeDtypeStruct((b, s, d), x.dtype),
    )(x, pe)


# speedup vs baseline: 2.0949x; 1.0693x over previous
import jax
import jax.numpy as jnp
from jax.experimental import pallas as pl


def _add_kernel(x_ref, pe_ref, o_ref):
    o_ref[...] = x_ref[...] + pe_ref[...]


def kernel(x, pos_emb):
    b, s, d = x.shape
    pe = pos_emb[:s]
    s_blk = 2048
    grid = (s // s_blk, b)
    return pl.pallas_call(
        _add_kernel,
        grid=grid,
        in_specs=[
            pl.BlockSpec((1, s_blk, d), lambda i, j: (j, i, 0)),
            pl.BlockSpec((s_blk, d), lambda i, j: (i, 0)),
        ],
        out_specs=pl.BlockSpec((1, s_blk, d), lambda i, j: (j, i, 0)),
        out_shape=jax.ShapeDtypeStruct((b, s, d), x.dtype),
    )(x, pe)
